# baseline (speedup 1.0000x reference)
import functools

import jax
import jax.numpy as jnp
from jax import lax
from jax.experimental import pallas as pl
from jax.experimental.pallas import tpu as pltpu
from jax.experimental.pallas import tpu_sc as plsc

_N = 1048576
_T = 1000
_NC = 2
_NS = 16
_NW = _NC * _NS
_PER_W = _N // _NW
_CHUNK = 2048
_NCHUNK = _PER_W // _CHUNK
_L = 16


def _popart_body(values_hbm, ids_hbm, w_hbm, b_hbm, s_hbm, m_hbm,
                 out_n_hbm, out_u_hbm,
                 w_v, b_v, s_v, m_v,
                 ids0, ids1, vals0, vals1, on0, on1, ou0, ou1,
                 sin0, sin1, sout0, sout1, tsem):
    wid = lax.axis_index("s") * _NC + lax.axis_index("c")
    base = wid * _PER_W

    ids_bufs = [ids0, ids1]
    vals_bufs = [vals0, vals1]
    on_bufs = [on0, on1]
    ou_bufs = [ou0, ou1]
    sin = [sin0, sin1]
    sout = [sout0, sout1]

    th = [
        pltpu.async_copy(w_hbm, w_v, tsem),
        pltpu.async_copy(b_hbm, b_v, tsem),
        pltpu.async_copy(s_hbm, s_v, tsem),
        pltpu.async_copy(m_hbm, m_v, tsem),
    ]

    def start_in(ci):
        bi = ci % 2
        off = base + ci * _CHUNK
        h1 = pltpu.async_copy(ids_hbm.at[pl.ds(off, _CHUNK)], ids_bufs[bi], sin[bi])
        h2 = pltpu.async_copy(values_hbm.at[pl.ds(off, _CHUNK)], vals_bufs[bi], sin[bi])
        return (h1, h2)

    in_h = [None] * _NCHUNK
    out_h = [None] * _NCHUNK
    in_h[0] = start_in(0)
    for h in th:
        h.wait()

    for ci in range(_NCHUNK):
        bi = ci % 2
        if ci + 1 < _NCHUNK:
            in_h[ci + 1] = start_in(ci + 1)
        in_h[ci][0].wait()
        in_h[ci][1].wait()
        if ci >= 2:
            out_h[ci - 2][0].wait()
            out_h[ci - 2][1].wait()

        iv, vv = ids_bufs[bi], vals_bufs[bi]
        onv, ouv = on_bufs[bi], ou_bufs[bi]

        @plsc.parallel_loop(0, _CHUNK // _L, unroll=8)
        def vec_body(j):
            sl = pl.ds(j * _L, _L)
            tid = iv[sl]
            xv = vv[sl]
            wv = plsc.load_gather(w_v, [tid])
            bv = plsc.load_gather(b_v, [tid])
            sv = plsc.load_gather(s_v, [tid])
            mv = plsc.load_gather(m_v, [tid])
            nv = wv * xv + bv
            onv[sl] = nv
            ouv[sl] = sv * nv + mv

        off = base + ci * _CHUNK
        out_h[ci] = (
            pltpu.async_copy(onv, out_n_hbm.at[pl.ds(off, _CHUNK)], sout[bi]),
            pltpu.async_copy(ouv, out_u_hbm.at[pl.ds(off, _CHUNK)], sout[bi]),
        )

    for ci in range(max(0, _NCHUNK - 2), _NCHUNK):
        out_h[ci][0].wait()
        out_h[ci][1].wait()


@jax.jit
def kernel(values, task_ids, w, b, sigma, mu):
    mesh = plsc.VectorSubcoreMesh(core_axis_name="c", subcore_axis_name="s")
    f = pl.kernel(
        _popart_body,
        mesh=mesh,
        out_type=[
            jax.ShapeDtypeStruct((_N,), jnp.float32),
            jax.ShapeDtypeStruct((_N,), jnp.float32),
        ],
        scratch_types=[
            pltpu.VMEM((_T,), jnp.float32),
            pltpu.VMEM((_T,), jnp.float32),
            pltpu.VMEM((_T,), jnp.float32),
            pltpu.VMEM((_T,), jnp.float32),
            pltpu.VMEM((_CHUNK,), jnp.int32),
            pltpu.VMEM((_CHUNK,), jnp.int32),
            pltpu.VMEM((_CHUNK,), jnp.float32),
            pltpu.VMEM((_CHUNK,), jnp.float32),
            pltpu.VMEM((_CHUNK,), jnp.float32),
            pltpu.VMEM((_CHUNK,), jnp.float32),
            pltpu.VMEM((_CHUNK,), jnp.float32),
            pltpu.VMEM((_CHUNK,), jnp.float32),
            pltpu.SemaphoreType.DMA,
            pltpu.SemaphoreType.DMA,
            pltpu.SemaphoreType.DMA,
            pltpu.SemaphoreType.DMA,
            pltpu.SemaphoreType.DMA,
        ],
        compiler_params=pltpu.CompilerParams(needs_layout_passes=False),
    )
    out_n, out_u = f(values, task_ids, w, b, sigma, mu)
    return (out_n, out_u)


# speedup vs baseline: 1.1721x; 1.1721x over previous
import functools

import jax
import jax.numpy as jnp
from jax import lax
from jax.experimental import pallas as pl
from jax.experimental.pallas import tpu as pltpu
from jax.experimental.pallas import tpu_sc as plsc

_N = 1048576
_T = 1000
_NC = 2
_NS = 16
_NW = _NC * _NS
_PER_W = _N // _NW
_CHUNK = 8192
_NCHUNK = _PER_W // _CHUNK
_L = 16


def _popart_body(values_hbm, ids_hbm, w_hbm, b_hbm, s_hbm, m_hbm,
                 out_n_hbm, out_u_hbm,
                 w_v, b_v, s_v, m_v,
                 ids0, ids1, vals0, vals1, on0, on1, ou0, ou1,
                 sin0, sin1, sout0, sout1, tsem):
    wid = lax.axis_index("s") * _NC + lax.axis_index("c")
    base = wid * _PER_W

    ids_bufs = [ids0, ids1]
    vals_bufs = [vals0, vals1]
    on_bufs = [on0, on1]
    ou_bufs = [ou0, ou1]
    sin = [sin0, sin1]
    sout = [sout0, sout1]

    th = [
        pltpu.async_copy(w_hbm, w_v, tsem),
        pltpu.async_copy(b_hbm, b_v, tsem),
        pltpu.async_copy(s_hbm, s_v, tsem),
        pltpu.async_copy(m_hbm, m_v, tsem),
    ]

    def start_in(ci):
        bi = ci % 2
        off = base + ci * _CHUNK
        h1 = pltpu.async_copy(ids_hbm.at[pl.ds(off, _CHUNK)], ids_bufs[bi], sin[bi])
        h2 = pltpu.async_copy(values_hbm.at[pl.ds(off, _CHUNK)], vals_bufs[bi], sin[bi])
        return (h1, h2)

    in_h = [None] * _NCHUNK
    out_h = [None] * _NCHUNK
    in_h[0] = start_in(0)
    for h in th:
        h.wait()

    for ci in range(_NCHUNK):
        bi = ci % 2
        if ci + 1 < _NCHUNK:
            in_h[ci + 1] = start_in(ci + 1)
        in_h[ci][0].wait()
        in_h[ci][1].wait()
        if ci >= 2:
            out_h[ci - 2][0].wait()
            out_h[ci - 2][1].wait()

        iv, vv = ids_bufs[bi], vals_bufs[bi]
        onv, ouv = on_bufs[bi], ou_bufs[bi]

        @plsc.parallel_loop(0, _CHUNK // _L, unroll=8)
        def vec_body(j):
            sl = pl.ds(j * _L, _L)
            tid = iv[sl]
            xv = vv[sl]
            wv = plsc.load_gather(w_v, [tid])
            bv = plsc.load_gather(b_v, [tid])
            sv = plsc.load_gather(s_v, [tid])
            mv = plsc.load_gather(m_v, [tid])
            nv = wv * xv + bv
            onv[sl] = nv
            ouv[sl] = sv * nv + mv

        off = base + ci * _CHUNK
        out_h[ci] = (
            pltpu.async_copy(onv, out_n_hbm.at[pl.ds(off, _CHUNK)], sout[bi]),
            pltpu.async_copy(ouv, out_u_hbm.at[pl.ds(off, _CHUNK)], sout[bi]),
        )

    for ci in range(max(0, _NCHUNK - 2), _NCHUNK):
        out_h[ci][0].wait()
        out_h[ci][1].wait()


@jax.jit
def kernel(values, task_ids, w, b, sigma, mu):
    mesh = plsc.VectorSubcoreMesh(core_axis_name="c", subcore_axis_name="s")
    f = pl.kernel(
        _popart_body,
        mesh=mesh,
        out_type=[
            jax.ShapeDtypeStruct((_N,), jnp.float32),
            jax.ShapeDtypeStruct((_N,), jnp.float32),
        ],
        scratch_types=[
            pltpu.VMEM((_T,), jnp.float32),
            pltpu.VMEM((_T,), jnp.float32),
            pltpu.VMEM((_T,), jnp.float32),
            pltpu.VMEM((_T,), jnp.float32),
            pltpu.VMEM((_CHUNK,), jnp.int32),
            pltpu.VMEM((_CHUNK,), jnp.int32),
            pltpu.VMEM((_CHUNK,), jnp.float32),
            pltpu.VMEM((_CHUNK,), jnp.float32),
            pltpu.VMEM((_CHUNK,), jnp.float32),
            pltpu.VMEM((_CHUNK,), jnp.float32),
            pltpu.VMEM((_CHUNK,), jnp.float32),
            pltpu.VMEM((_CHUNK,), jnp.float32),
            pltpu.SemaphoreType.DMA,
            pltpu.SemaphoreType.DMA,
            pltpu.SemaphoreType.DMA,
            pltpu.SemaphoreType.DMA,
            pltpu.SemaphoreType.DMA,
        ],
        compiler_params=pltpu.CompilerParams(needs_layout_passes=False),
    )
    out_n, out_u = f(values, task_ids, w, b, sigma, mu)
    return (out_n, out_u)
